# baseline (speedup 1.0000x reference)
import jax
import jax.numpy as jnp
from jax.experimental import pallas as pl
from jax.experimental.pallas import tpu as pltpu

_ROWS = 100000
_D = 128
_K = 8

_T = _ROWS // 8
_SIZES = [(_T // _K + (1 if k < _T % _K else 0)) * 8 for k in range(_K)]
_BASES = [sum(_SIZES[:k]) for k in range(_K)]


def _body(xs_ref, x_ref, out_ref, *sems):
    cps = []
    for k in range(_K):
        cp = pltpu.make_async_copy(
            xs_ref.at[pl.ds(_BASES[k], _SIZES[k])],
            out_ref.at[pl.ds(_BASES[k], _SIZES[k])],
            sems[k],
        )
        cp.start()
        cps.append(cp)
    cps[0].wait()
    xcp = pltpu.make_async_copy(x_ref, out_ref.at[pl.ds(0, 1)], sems[_K])
    xcp.start()
    for cp in cps[1:]:
        cp.wait()
    xcp.wait()


@jax.jit
def kernel(xs, x):
    return pl.pallas_call(
        _body,
        out_shape=jax.ShapeDtypeStruct((_ROWS, _D), jnp.float32),
        in_specs=[
            pl.BlockSpec(memory_space=pl.ANY),
            pl.BlockSpec(memory_space=pl.ANY),
        ],
        out_specs=pl.BlockSpec(memory_space=pl.ANY),
        scratch_shapes=[pltpu.SemaphoreType.DMA] * (_K + 1),
    )(xs, x)


# speedup vs baseline: 44.6461x; 44.6461x over previous
import jax
import jax.numpy as jnp
from jax.experimental import pallas as pl
from jax.experimental.pallas import tpu as pltpu

_ROWS = 100000
_D = 128
_BS = 5000
_GRID = _ROWS // _BS


def _body(xs_ref, x_ref, out_ref):
    out_ref[...] = xs_ref[...]

    @pl.when(pl.program_id(0) == 0)
    def _():
        out_ref[0:1, :] = x_ref[...]


@jax.jit
def kernel(xs, x):
    return pl.pallas_call(
        _body,
        grid=(_GRID,),
        out_shape=jax.ShapeDtypeStruct((_ROWS, _D), jnp.float32),
        in_specs=[
            pl.BlockSpec((_BS, _D), lambda i: (i, 0)),
            pl.BlockSpec((1, _D), lambda i: (0, 0)),
        ],
        out_specs=pl.BlockSpec((_BS, _D), lambda i: (i, 0)),
        compiler_params=pltpu.CompilerParams(
            dimension_semantics=("arbitrary",),
        ),
    )(xs, x)
